# baseline (speedup 1.0000x reference)
import functools

import jax
import jax.numpy as jnp
from jax import lax
from jax.experimental import pallas as pl
from jax.experimental.pallas import tpu as pltpu
from jax.experimental.pallas import tpu_sc as plsc

B = 16384
F = 26
D = 16
V = 1000000
TW = 8192
TGRID = (V + TW - 1) // TW
VPAD = TGRID * TW

_INFO = plsc.get_sparse_core_info()
NC = _INFO.num_cores
NS = _INFO.num_subcores
NW = NC * NS

CB = 64
ROWS = CB * F
G = ROWS // 128
B_PER_W = B // NW
NCHUNK = B_PER_W // CB
IDX_ROWS = B * F // 128
IDX_BLK = 32
IDX_PAD = IDX_BLK * TGRID


def _transpose_body(x_ref, i_ref, o_ref, oi_ref):
    x = x_ref[...]
    x6 = (
        x.reshape(D, TW // 128, 128)
        .transpose(1, 0, 2)
        .reshape(TW // 1024, 128, 128)
    )
    y = x6.transpose(0, 2, 1)
    o_ref[...] = y.reshape(TW // 8, 128)
    t = i_ref[...]
    oi_ref[...] = ((t >> 10) << 10) + ((t & 127) << 3) + ((t >> 7) & 7)


def _transpose_table(wt, idx2d):
    return pl.pallas_call(
        _transpose_body,
        grid=(TGRID,),
        in_specs=[
            pl.BlockSpec((D, TW), lambda j: (0, j)),
            pl.BlockSpec((IDX_BLK, 128), lambda j: (j, 0)),
        ],
        out_specs=[
            pl.BlockSpec((TW // 8, 128), lambda j: (j, 0)),
            pl.BlockSpec((IDX_BLK, 128), lambda j: (j, 0)),
        ],
        out_shape=[
            jax.ShapeDtypeStruct((VPAD * D // 128, 128), jnp.float32),
            jax.ShapeDtypeStruct((IDX_PAD, 128), jnp.int32),
        ],
    )(wt, idx2d)


def _make_kernel():
    mesh = plsc.VectorSubcoreMesh(core_axis_name="c", subcore_axis_name="s")

    @functools.partial(
        pl.kernel,
        mesh=mesh,
        out_type=jax.ShapeDtypeStruct((B, D), jnp.float32),
        name="feature_linear_sc",
        scratch_types=[
            pltpu.VMEM((3, G, 128), jnp.int32),
            pltpu.VMEM((3, CB, F), jnp.float32),
            pltpu.VMEM((2, ROWS, D), jnp.float32),
            pltpu.VMEM((CB, D), jnp.float32),
            pltpu.VMEM((D,), jnp.float32),
            pltpu.SemaphoreType.DMA,
            pltpu.SemaphoreType.DMA,
            pltpu.SemaphoreType.DMA,
        ],
        compiler_params=pltpu.CompilerParams(use_tc_tiling_on_sc=False),
    )
    def feature_linear(idx_hbm, val_hbm, table_hbm, bias_hbm, out_hbm,
                       idx_v, val_v, rows_v, out_v, bias_v,
                       sem_in, semg0, semg1):
        wid = lax.axis_index("s") * NC + lax.axis_index("c")
        pltpu.sync_copy(bias_hbm, bias_v)
        semg = (semg0, semg1)

        def start_stage(c):
            p = c % 3
            base = wid * B_PER_W + c * CB
            goff = base * F // 128
            return (
                pltpu.async_copy(idx_hbm.at[pl.ds(goff, G)], idx_v.at[p],
                                 sem_in),
                pltpu.async_copy(val_hbm.at[pl.ds(base, CB)], val_v.at[p],
                                 sem_in),
            )

        def fire_gathers(c):
            p = c % 2
            return [
                pltpu.async_copy(
                    table_hbm.at[idx_v.at[c % 3, g]],
                    rows_v.at[p, pl.ds(g * 128, 128)],
                    semg[p],
                )
                for g in range(G)
            ]

        def compute(c):
            p = c % 2
            base = wid * B_PER_W + c * CB

            pv = c % 3

            def body(i, _):
                rb = i * F
                vlo = val_v[pv, i, 0:16]
                vhi = val_v[pv, i, F - 16:F]
                accs = [None] * 4
                for f in range(F):
                    v = vlo[f] if f < 16 else vhi[f - (F - 16)]
                    term = v * rows_v[p, rb + f, :]
                    k = f & 3
                    accs[k] = term if accs[k] is None else accs[k] + term
                out_v[i, :] = ((accs[0] + accs[1]) + (accs[2] + accs[3])
                               + bias_v[:])
                return 0

            lax.fori_loop(0, CB, body, 0)
            pltpu.sync_copy(out_v, out_hbm.at[pl.ds(base, CB)])

        stage = start_stage(0)
        for cp in stage:
            cp.wait()
        pending_gathers = fire_gathers(0)
        stage = start_stage(1)
        for c in range(NCHUNK):
            if c + 1 < NCHUNK:
                for cp in stage:
                    cp.wait()
                next_gathers = fire_gathers(c + 1)
            if c + 2 < NCHUNK:
                stage = start_stage(c + 2)
            for cp in pending_gathers:
                cp.wait()
            compute(c)
            if c + 1 < NCHUNK:
                pending_gathers = next_gathers

    return feature_linear


_kernel_fn = _make_kernel()


@jax.jit
def kernel(feature_idx, feature_value, weight, bias):
    idx2d = feature_idx.reshape(B * F // 128, 128)
    w_rm, idx_rm = _transpose_table(weight.T, idx2d)
    return _kernel_fn(idx_rm, feature_value, w_rm.reshape(VPAD, D), bias)


# speedup vs baseline: 1.0803x; 1.0803x over previous
import functools

import jax
import jax.numpy as jnp
from jax import lax
from jax.experimental import pallas as pl
from jax.experimental.pallas import tpu as pltpu
from jax.experimental.pallas import tpu_sc as plsc

B = 16384
F = 26
D = 16
V = 1000000
TW = 8192
TGRID = (V + TW - 1) // TW
VPAD = TGRID * TW

_INFO = plsc.get_sparse_core_info()
NC = _INFO.num_cores
NS = _INFO.num_subcores
NW = NC * NS

CB = 64
ROWS = CB * F
G = ROWS // 128
B_PER_W = B // NW
NCHUNK = B_PER_W // CB
IDX_ROWS = B * F // 128
IDX_BLK = 32
IDX_PAD = IDX_BLK * TGRID


def _transpose_body(x_ref, o_ref):
    x = x_ref[...]
    x6 = (
        x.reshape(D, TW // 128, 128)
        .transpose(1, 0, 2)
        .reshape(TW // 1024, 128, 128)
    )
    y = x6.transpose(0, 2, 1)
    o_ref[...] = y.reshape(TW // 8, 128)


def _transpose_table(wt):
    return pl.pallas_call(
        _transpose_body,
        grid=(TGRID,),
        in_specs=[pl.BlockSpec((D, TW), lambda j: (0, j))],
        out_specs=pl.BlockSpec((TW // 8, 128), lambda j: (j, 0)),
        out_shape=jax.ShapeDtypeStruct((VPAD * D // 128, 128), jnp.float32),
    )(wt)


def _make_kernel():
    mesh = plsc.VectorSubcoreMesh(core_axis_name="c", subcore_axis_name="s")

    @functools.partial(
        pl.kernel,
        mesh=mesh,
        out_type=jax.ShapeDtypeStruct((B, D), jnp.float32),
        name="feature_linear_sc",
        scratch_types=[
            pltpu.VMEM((3, F, CB), jnp.int32),
            pltpu.VMEM((3, ROWS), jnp.float32),
            pltpu.VMEM((2, ROWS, D), jnp.float32),
            pltpu.VMEM((CB, D), jnp.float32),
            pltpu.VMEM((D,), jnp.float32),
            pltpu.SemaphoreType.DMA,
            pltpu.SemaphoreType.DMA,
            pltpu.SemaphoreType.DMA,
        ],
        compiler_params=pltpu.CompilerParams(use_tc_tiling_on_sc=False),
    )
    def feature_linear(idx_hbm, val_hbm, table_hbm, bias_hbm, out_hbm,
                       idx_v, val_v, rows_v, out_v, bias_v,
                       sem_in, semg0, semg1):
        wid = lax.axis_index("s") * NC + lax.axis_index("c")
        pltpu.sync_copy(bias_hbm, bias_v)
        semg = (semg0, semg1)

        def start_stage(c):
            p = c % 3
            base = wid * B_PER_W + c * CB
            return (
                pltpu.async_copy(idx_hbm.at[:, pl.ds(base, CB)],
                                 idx_v.at[p], sem_in),
                pltpu.async_copy(val_hbm.at[pl.ds(base * F, ROWS)],
                                 val_v.at[p], sem_in),
            )

        def remap_and_fire(c):
            p = c % 3
            pr = c % 2

            def remap(i, _):
                f = i // (CB // 16)
                l = (i % (CB // 16)) * 16
                t = idx_v[p, f, pl.ds(l, 16)]
                g = ((t >> 10) << 10) + ((t & 127) << 3) + ((t >> 7) & 7)
                idx_v[p, f, pl.ds(l, 16)] = g
                return 0

            lax.fori_loop(0, F * (CB // 16), remap, 0)
            return [
                pltpu.async_copy(
                    table_hbm.at[idx_v.at[p, f]],
                    rows_v.at[pr, pl.ds(f * CB, CB)],
                    semg[pr],
                )
                for f in range(F)
            ]

        def compute(c):
            p = c % 2
            pv = c % 3
            base = wid * B_PER_W + c * CB

            def body(i, _):
                fb = i * F
                vlo = val_v[pv, pl.ds(fb, 16)]
                vhi = val_v[pv, pl.ds(fb + (F - 16), 16)]
                accs = [None] * 4
                for f in range(F):
                    v = vlo[f] if f < 16 else vhi[f - (F - 16)]
                    term = v * rows_v[p, f * CB + i, :]
                    k = f & 3
                    accs[k] = term if accs[k] is None else accs[k] + term
                out_v[i, :] = ((accs[0] + accs[1]) + (accs[2] + accs[3])
                               + bias_v[:])
                return 0

            lax.fori_loop(0, CB, body, 0)
            pltpu.sync_copy(out_v, out_hbm.at[pl.ds(base, CB)])

        stage = start_stage(0)
        for cp in stage:
            cp.wait()
        pending_gathers = remap_and_fire(0)
        stage = start_stage(1)
        for c in range(NCHUNK):
            if c + 1 < NCHUNK:
                for cp in stage:
                    cp.wait()
                next_gathers = remap_and_fire(c + 1)
            if c + 2 < NCHUNK:
                stage = start_stage(c + 2)
            for cp in pending_gathers:
                cp.wait()
            compute(c)
            if c + 1 < NCHUNK:
                pending_gathers = next_gathers

    return feature_linear


_kernel_fn = _make_kernel()


@jax.jit
def kernel(feature_idx, feature_value, weight, bias):
    w_rm = _transpose_table(weight.T).reshape(VPAD, D)
    return _kernel_fn(feature_idx.T, feature_value.reshape(B * F), w_rm, bias)


# speedup vs baseline: 1.3066x; 1.2095x over previous
import functools

import jax
import jax.numpy as jnp
from jax import lax
from jax.experimental import pallas as pl
from jax.experimental.pallas import tpu as pltpu
from jax.experimental.pallas import tpu_sc as plsc

B = 16384
F = 26
D = 16
V = 1000000
TW = 16384
TGRID = (V + TW - 1) // TW
VPAD = TGRID * TW

_INFO = plsc.get_sparse_core_info()
NC = _INFO.num_cores
NS = _INFO.num_subcores
NW = NC * NS

CB = 64
ROWS = CB * F
G = ROWS // 128
B_PER_W = B // NW
NCHUNK = B_PER_W // CB
IDX_ROWS = B * F // 128
IDX_BLK = 32
IDX_PAD = IDX_BLK * TGRID


def _transpose_body(x_ref, o_ref):
    x = x_ref[...]
    x6 = (
        x.reshape(D, TW // 128, 128)
        .transpose(1, 0, 2)
        .reshape(TW // 1024, 128, 128)
    )
    y = x6.transpose(0, 2, 1)
    o_ref[...] = y.reshape(TW // 8, 128)


def _transpose_table(wt):
    return pl.pallas_call(
        _transpose_body,
        grid=(TGRID,),
        in_specs=[pl.BlockSpec((D, TW), lambda j: (0, j))],
        out_specs=pl.BlockSpec((TW // 8, 128), lambda j: (j, 0)),
        out_shape=jax.ShapeDtypeStruct((VPAD * D // 128, 128), jnp.float32),
    )(wt)


def _make_kernel():
    mesh = plsc.VectorSubcoreMesh(core_axis_name="c", subcore_axis_name="s")

    @functools.partial(
        pl.kernel,
        mesh=mesh,
        out_type=jax.ShapeDtypeStruct((B, D), jnp.float32),
        name="feature_linear_sc",
        scratch_types=[
            pltpu.VMEM((3, F, CB), jnp.int32),
            pltpu.VMEM((3, ROWS), jnp.float32),
            pltpu.VMEM((2, ROWS, D), jnp.float32),
            pltpu.VMEM((CB, D), jnp.float32),
            pltpu.VMEM((D,), jnp.float32),
            pltpu.SemaphoreType.DMA,
            pltpu.SemaphoreType.DMA,
            pltpu.SemaphoreType.DMA,
        ],
        compiler_params=pltpu.CompilerParams(use_tc_tiling_on_sc=False),
    )
    def feature_linear(idx_hbm, val_hbm, table_hbm, bias_hbm, out_hbm,
                       idx_v, val_v, rows_v, out_v, bias_v,
                       sem_in, semg0, semg1):
        wid = lax.axis_index("s") * NC + lax.axis_index("c")
        pltpu.sync_copy(bias_hbm, bias_v)
        semg = (semg0, semg1)

        def start_stage(c):
            p = c % 3
            base = wid * B_PER_W + c * CB
            return (
                pltpu.async_copy(idx_hbm.at[:, pl.ds(base, CB)],
                                 idx_v.at[p], sem_in),
                pltpu.async_copy(val_hbm.at[pl.ds(base * F, ROWS)],
                                 val_v.at[p], sem_in),
            )

        def remap_and_fire(c):
            p = c % 3
            pr = c % 2

            def remap(i, _):
                f = i // (CB // 16)
                l = (i % (CB // 16)) * 16
                t = idx_v[p, f, pl.ds(l, 16)]
                g = ((t >> 10) << 10) + ((t & 127) << 3) + ((t >> 7) & 7)
                idx_v[p, f, pl.ds(l, 16)] = g
                return 0

            lax.fori_loop(0, F * (CB // 16), remap, 0)
            return [
                pltpu.async_copy(
                    table_hbm.at[idx_v.at[p, f]],
                    rows_v.at[pr, pl.ds(f * CB, CB)],
                    semg[pr],
                )
                for f in range(F)
            ]

        def compute(c):
            p = c % 2
            pv = c % 3
            base = wid * B_PER_W + c * CB

            def body(i, _):
                fb = i * F
                vlo = val_v[pv, pl.ds(fb, 16)]
                vhi = val_v[pv, pl.ds(fb + (F - 16), 16)]
                accs = [None] * 4
                for f in range(F):
                    v = vlo[f] if f < 16 else vhi[f - (F - 16)]
                    term = v * rows_v[p, f * CB + i, :]
                    k = f & 3
                    accs[k] = term if accs[k] is None else accs[k] + term
                out_v[i, :] = ((accs[0] + accs[1]) + (accs[2] + accs[3])
                               + bias_v[:])
                return 0

            lax.fori_loop(0, CB, body, 0)
            pltpu.sync_copy(out_v, out_hbm.at[pl.ds(base, CB)])

        stage = start_stage(0)
        for cp in stage:
            cp.wait()
        pending_gathers = remap_and_fire(0)
        stage = start_stage(1)
        for c in range(NCHUNK):
            if c + 1 < NCHUNK:
                for cp in stage:
                    cp.wait()
                next_gathers = remap_and_fire(c + 1)
            if c + 2 < NCHUNK:
                stage = start_stage(c + 2)
            for cp in pending_gathers:
                cp.wait()
            compute(c)
            if c + 1 < NCHUNK:
                pending_gathers = next_gathers

    return feature_linear


_kernel_fn = _make_kernel()


@jax.jit
def kernel(feature_idx, feature_value, weight, bias):
    w_rm = _transpose_table(weight.T).reshape(VPAD, D)
    return _kernel_fn(feature_idx.T, feature_value.reshape(B * F), w_rm, bias)


# speedup vs baseline: 1.5324x; 1.1728x over previous
import functools

import jax
import jax.numpy as jnp
from jax import lax
from jax.experimental import pallas as pl
from jax.experimental.pallas import tpu as pltpu
from jax.experimental.pallas import tpu_sc as plsc

B = 16384
F = 26
D = 16
V = 1000000
TW = 40960
TGRID = (V + TW - 1) // TW
VPAD = TGRID * TW

_INFO = plsc.get_sparse_core_info()
NC = _INFO.num_cores
NS = _INFO.num_subcores
NW = NC * NS

CB = 64
ROWS = CB * F
G = ROWS // 128
B_PER_W = B // NW
NCHUNK = B_PER_W // CB
IDX_ROWS = B * F // 128
IDX_BLK = 32
IDX_PAD = IDX_BLK * TGRID


def _transpose_body(x_ref, o_ref):
    x = x_ref[...]
    x6 = (
        x.reshape(D, TW // 128, 128)
        .transpose(1, 0, 2)
        .reshape(TW // 1024, 128, 128)
    )
    y = x6.transpose(0, 2, 1)
    o_ref[...] = y.reshape(TW // 8, 128)


def _transpose_table(wt):
    return pl.pallas_call(
        _transpose_body,
        grid=(TGRID,),
        in_specs=[pl.BlockSpec((D, TW), lambda j: (0, j))],
        out_specs=pl.BlockSpec((TW // 8, 128), lambda j: (j, 0)),
        out_shape=jax.ShapeDtypeStruct((VPAD * D // 128, 128), jnp.float32),
    )(wt)


def _make_kernel():
    mesh = plsc.VectorSubcoreMesh(core_axis_name="c", subcore_axis_name="s")

    @functools.partial(
        pl.kernel,
        mesh=mesh,
        out_type=jax.ShapeDtypeStruct((B, D), jnp.float32),
        name="feature_linear_sc",
        scratch_types=[
            pltpu.VMEM((3, F, CB), jnp.int32),
            pltpu.VMEM((3, ROWS), jnp.float32),
            pltpu.VMEM((2, ROWS, D), jnp.float32),
            pltpu.VMEM((CB, D), jnp.float32),
            pltpu.VMEM((D,), jnp.float32),
            pltpu.SemaphoreType.DMA,
            pltpu.SemaphoreType.DMA,
            pltpu.SemaphoreType.DMA,
        ],
        compiler_params=pltpu.CompilerParams(use_tc_tiling_on_sc=False),
    )
    def feature_linear(idx_hbm, val_hbm, table_hbm, bias_hbm, out_hbm,
                       idx_v, val_v, rows_v, out_v, bias_v,
                       sem_in, semg0, semg1):
        wid = lax.axis_index("s") * NC + lax.axis_index("c")
        pltpu.sync_copy(bias_hbm, bias_v)
        semg = (semg0, semg1)

        def start_stage(c):
            p = c % 3
            base = wid * B_PER_W + c * CB
            return (
                pltpu.async_copy(idx_hbm.at[:, pl.ds(base, CB)],
                                 idx_v.at[p], sem_in),
                pltpu.async_copy(val_hbm.at[pl.ds(base * F, ROWS)],
                                 val_v.at[p], sem_in),
            )

        def remap_and_fire(c):
            p = c % 3
            pr = c % 2

            def remap(i, _):
                f = i // (CB // 16)
                l = (i % (CB // 16)) * 16
                t = idx_v[p, f, pl.ds(l, 16)]
                g = ((t >> 10) << 10) + ((t & 127) << 3) + ((t >> 7) & 7)
                idx_v[p, f, pl.ds(l, 16)] = g
                return 0

            lax.fori_loop(0, F * (CB // 16), remap, 0)
            return [
                pltpu.async_copy(
                    table_hbm.at[idx_v.at[p, f]],
                    rows_v.at[pr, pl.ds(f * CB, CB)],
                    semg[pr],
                )
                for f in range(F)
            ]

        def compute(c):
            p = c % 2
            pv = c % 3
            base = wid * B_PER_W + c * CB

            def body(i, _):
                fb = i * F
                vlo = val_v[pv, pl.ds(fb, 16)]
                vhi = val_v[pv, pl.ds(fb + (F - 16), 16)]
                accs = [None] * 4
                for f in range(F):
                    v = vlo[f] if f < 16 else vhi[f - (F - 16)]
                    term = v * rows_v[p, f * CB + i, :]
                    k = f & 3
                    accs[k] = term if accs[k] is None else accs[k] + term
                out_v[i, :] = ((accs[0] + accs[1]) + (accs[2] + accs[3])
                               + bias_v[:])
                return 0

            lax.fori_loop(0, CB, body, 0)
            pltpu.sync_copy(out_v, out_hbm.at[pl.ds(base, CB)])

        stage = start_stage(0)
        for cp in stage:
            cp.wait()
        pending_gathers = remap_and_fire(0)
        stage = start_stage(1)
        for c in range(NCHUNK):
            if c + 1 < NCHUNK:
                for cp in stage:
                    cp.wait()
                next_gathers = remap_and_fire(c + 1)
            if c + 2 < NCHUNK:
                stage = start_stage(c + 2)
            for cp in pending_gathers:
                cp.wait()
            compute(c)
            if c + 1 < NCHUNK:
                pending_gathers = next_gathers

    return feature_linear


_kernel_fn = _make_kernel()


@jax.jit
def kernel(feature_idx, feature_value, weight, bias):
    w_rm = _transpose_table(weight.T).reshape(VPAD, D)
    return _kernel_fn(feature_idx.T, feature_value.reshape(B * F), w_rm, bias)


# speedup vs baseline: 1.5866x; 1.0354x over previous
import functools

import jax
import jax.numpy as jnp
from jax import lax
from jax.experimental import pallas as pl
from jax.experimental.pallas import tpu as pltpu
from jax.experimental.pallas import tpu_sc as plsc

B = 16384
F = 26
D = 16
V = 1000000
TW = 71680
TGRID = (V + TW - 1) // TW
VPAD = TGRID * TW

_INFO = plsc.get_sparse_core_info()
NC = _INFO.num_cores
NS = _INFO.num_subcores
NW = NC * NS

CB = 64
ROWS = CB * F
G = ROWS // 128
B_PER_W = B // NW
NCHUNK = B_PER_W // CB
IDX_ROWS = B * F // 128
IDX_BLK = 32
IDX_PAD = IDX_BLK * TGRID


def _transpose_body(x_ref, o_ref):
    x = x_ref[...]
    x6 = (
        x.reshape(D, TW // 128, 128)
        .transpose(1, 0, 2)
        .reshape(TW // 1024, 128, 128)
    )
    y = x6.transpose(0, 2, 1)
    o_ref[...] = y.reshape(TW // 8, 128)


def _transpose_table(wt):
    return pl.pallas_call(
        _transpose_body,
        grid=(TGRID,),
        in_specs=[pl.BlockSpec((D, TW), lambda j: (0, j))],
        out_specs=pl.BlockSpec((TW // 8, 128), lambda j: (j, 0)),
        out_shape=jax.ShapeDtypeStruct((VPAD * D // 128, 128), jnp.float32),
    )(wt)


def _make_kernel():
    mesh = plsc.VectorSubcoreMesh(core_axis_name="c", subcore_axis_name="s")

    @functools.partial(
        pl.kernel,
        mesh=mesh,
        out_type=jax.ShapeDtypeStruct((B, D), jnp.float32),
        name="feature_linear_sc",
        scratch_types=[
            pltpu.VMEM((3, F, CB), jnp.int32),
            pltpu.VMEM((3, ROWS), jnp.float32),
            pltpu.VMEM((2, ROWS, D), jnp.float32),
            pltpu.VMEM((CB, D), jnp.float32),
            pltpu.VMEM((D,), jnp.float32),
            pltpu.SemaphoreType.DMA,
            pltpu.SemaphoreType.DMA,
            pltpu.SemaphoreType.DMA,
        ],
        compiler_params=pltpu.CompilerParams(use_tc_tiling_on_sc=False),
    )
    def feature_linear(idx_hbm, val_hbm, table_hbm, bias_hbm, out_hbm,
                       idx_v, val_v, rows_v, out_v, bias_v,
                       sem_in, semg0, semg1):
        wid = lax.axis_index("s") * NC + lax.axis_index("c")
        pltpu.sync_copy(bias_hbm, bias_v)
        semg = (semg0, semg1)

        def start_stage(c):
            p = c % 3
            base = wid * B_PER_W + c * CB
            return (
                pltpu.async_copy(idx_hbm.at[:, pl.ds(base, CB)],
                                 idx_v.at[p], sem_in),
                pltpu.async_copy(val_hbm.at[pl.ds(base * F, ROWS)],
                                 val_v.at[p], sem_in),
            )

        def remap_and_fire(c):
            p = c % 3
            pr = c % 2

            def remap(i, _):
                f = i // (CB // 16)
                l = (i % (CB // 16)) * 16
                t = idx_v[p, f, pl.ds(l, 16)]
                g = ((t >> 10) << 10) + ((t & 127) << 3) + ((t >> 7) & 7)
                idx_v[p, f, pl.ds(l, 16)] = g
                return 0

            lax.fori_loop(0, F * (CB // 16), remap, 0)
            return [
                pltpu.async_copy(
                    table_hbm.at[idx_v.at[p, f]],
                    rows_v.at[pr, pl.ds(f * CB, CB)],
                    semg[pr],
                )
                for f in range(F)
            ]

        def compute(c):
            p = c % 2
            pv = c % 3
            base = wid * B_PER_W + c * CB

            def body(i, _):
                fb = i * F
                vlo = val_v[pv, pl.ds(fb, 16)]
                vhi = val_v[pv, pl.ds(fb + (F - 16), 16)]
                accs = [None] * 4
                for f in range(F):
                    v = vlo[f] if f < 16 else vhi[f - (F - 16)]
                    term = v * rows_v[p, f * CB + i, :]
                    k = f & 3
                    accs[k] = term if accs[k] is None else accs[k] + term
                out_v[i, :] = ((accs[0] + accs[1]) + (accs[2] + accs[3])
                               + bias_v[:])
                return 0

            lax.fori_loop(0, CB, body, 0)
            pltpu.sync_copy(out_v, out_hbm.at[pl.ds(base, CB)])

        stage = start_stage(0)
        for cp in stage:
            cp.wait()
        pending_gathers = remap_and_fire(0)
        stage = start_stage(1)
        for c in range(NCHUNK):
            if c + 1 < NCHUNK:
                for cp in stage:
                    cp.wait()
                next_gathers = remap_and_fire(c + 1)
            if c + 2 < NCHUNK:
                stage = start_stage(c + 2)
            for cp in pending_gathers:
                cp.wait()
            compute(c)
            if c + 1 < NCHUNK:
                pending_gathers = next_gathers

    return feature_linear


_kernel_fn = _make_kernel()


@jax.jit
def kernel(feature_idx, feature_value, weight, bias):
    w_rm = _transpose_table(weight.T).reshape(VPAD, D)
    return _kernel_fn(feature_idx.T, feature_value.reshape(B * F), w_rm, bias)
